# baseline (device time: 16197 ns/iter reference)
import jax
import jax.numpy as jnp
from jax import lax
from jax.experimental import pallas as pl
from jax.experimental.pallas import tpu as pltpu

C = 8


def kernel(x):
    _, m, n = x.shape
    cols = n // 2
    hr = m // 2
    rc = hr // C

    def body(x_ref, out_ref, xrecv, zrecv, xs_sems, xr_sems, zs_sems, zr_sems):
        mx = lax.axis_index("x")
        my = lax.axis_index("y")
        mz = lax.axis_index("z")
        pz = lax.rem(mz, 2)
        partner = (1 - mx, my, mz)
        zmate = (mx, my, mz + 1 - 2 * pz)

        barrier_sem = pltpu.get_barrier_semaphore()
        for nbr in (partner, zmate):
            pl.semaphore_signal(
                barrier_sem, inc=1,
                device_id=nbr, device_id_type=pl.DeviceIdType.MESH,
            )
        pl.semaphore_wait(barrier_sem, 2)

        pcol0 = (1 - mx) * cols
        mcol0 = mx * cols
        row0 = pz * hr
        zrow0 = (1 - pz) * hr

        xr = []
        for k in range(C):
            r = pltpu.make_async_remote_copy(
                src_ref=x_ref.at[0, pl.ds(row0 + k * rc, rc), pl.ds(pcol0, cols)],
                dst_ref=xrecv.at[k],
                send_sem=xs_sems.at[k],
                recv_sem=xr_sems.at[k],
                device_id=partner,
                device_id_type=pl.DeviceIdType.MESH,
            )
            r.start()
            xr.append(r)

        zr = []
        for k in range(C):
            xr[k].wait_recv()
            f = pltpu.make_async_remote_copy(
                src_ref=xrecv.at[k],
                dst_ref=zrecv.at[k],
                send_sem=zs_sems.at[k],
                recv_sem=zr_sems.at[k],
                device_id=zmate,
                device_id_type=pl.DeviceIdType.MESH,
            )
            f.start()
            zr.append(f)
            out_ref[pl.ds(row0 + k * rc, rc), :] = (
                x_ref[0, pl.ds(row0 + k * rc, rc), pl.ds(mcol0, cols)]
                + xrecv[k]
            )

        for k in range(C):
            zr[k].wait_recv()
            out_ref[pl.ds(zrow0 + k * rc, rc), :] = (
                x_ref[0, pl.ds(zrow0 + k * rc, rc), pl.ds(mcol0, cols)]
                + zrecv[k]
            )

        for k in range(C):
            xr[k].wait_send()
            zr[k].wait_send()

    return pl.pallas_call(
        body,
        out_shape=jax.ShapeDtypeStruct((m, cols), jnp.float32),
        in_specs=[pl.BlockSpec(memory_space=pltpu.VMEM)],
        out_specs=pl.BlockSpec(memory_space=pltpu.VMEM),
        scratch_shapes=[
            pltpu.VMEM((C, rc, cols), jnp.float32),
            pltpu.VMEM((C, rc, cols), jnp.float32),
            pltpu.SemaphoreType.DMA((C,)),
            pltpu.SemaphoreType.DMA((C,)),
            pltpu.SemaphoreType.DMA((C,)),
            pltpu.SemaphoreType.DMA((C,)),
        ],
        compiler_params=pltpu.CompilerParams(collective_id=0),
    )(x)
